# baseline (speedup 1.0000x reference)
import functools

import jax
import jax.numpy as jnp
from jax.experimental import pallas as pl
from jax.experimental.pallas import tpu as pltpu

B = 8
N = 20000
C = 92
BN = 2000


def _body(scale_ref, logit_ref, box_ref, score_ref, label_ref, obox_ref,
          idx_ref):
    pid = pl.program_id(0)
    b = (pid * BN) // N

    x = logit_ref[0]
    m_all = jnp.max(x, axis=-1, keepdims=True)
    cols = jax.lax.broadcasted_iota(jnp.int32, (BN, C), 1)
    valid = cols < (C - 1)
    xm = jnp.where(valid, x, -jnp.inf)
    m91 = jnp.max(xm, axis=-1, keepdims=True)
    lbl = jnp.min(jnp.where(xm == m91, cols, C), axis=-1)
    denom = jnp.sum(jnp.exp(x - m_all), axis=-1, keepdims=True)
    score = jnp.exp(m91 - m_all) / denom

    score_ref[0] = score.reshape(1, BN)
    label_ref[0] = lbl.reshape(1, BN).astype(jnp.int32)

    w_s = scale_ref[b, 0]
    h_s = scale_ref[b, 1]
    wh_cols = jax.lax.broadcasted_iota(jnp.int32, (BN, 2), 1)
    wh = jnp.where(wh_cols == 0, w_s, h_s)
    bx = box_ref[0]
    ctr = bx[:, 0:2]
    half = 0.5 * bx[:, 2:4]
    lo = (ctr - half) * wh
    hi = (ctr + half) * wh
    obox_ref[0] = jnp.concatenate([lo, hi], axis=-1)

    base = (pid * BN) % N
    idx_ref[0] = (base + jax.lax.broadcasted_iota(
        jnp.int32, (1, BN), 1)).astype(jnp.float32)


def kernel(pred_logits, pred_boxes, target_sizes):
    nb = (B * N) // BN
    logits = pred_logits.reshape(nb, BN, C)
    boxes = pred_boxes.reshape(nb, BN, 4)
    ts = target_sizes.astype(jnp.float32)
    img_h = ts[:, 0]
    img_w = ts[:, 1]
    scale = jnp.stack([img_w, img_h, img_w, img_h], axis=1)

    grid = (nb,)
    out_shapes = (
        jax.ShapeDtypeStruct((nb, 1, BN), jnp.float32),
        jax.ShapeDtypeStruct((nb, 1, BN), jnp.int32),
        jax.ShapeDtypeStruct((nb, BN, 4), jnp.float32),
        jax.ShapeDtypeStruct((nb, 1, BN), jnp.float32),
    )
    in_specs = [
        pl.BlockSpec(memory_space=pltpu.SMEM),
        pl.BlockSpec((1, BN, C), lambda i: (i, 0, 0)),
        pl.BlockSpec((1, BN, 4), lambda i: (i, 0, 0)),
    ]
    out_specs = (
        pl.BlockSpec((1, 1, BN), lambda i: (i, 0, 0)),
        pl.BlockSpec((1, 1, BN), lambda i: (i, 0, 0)),
        pl.BlockSpec((1, BN, 4), lambda i: (i, 0, 0)),
        pl.BlockSpec((1, 1, BN), lambda i: (i, 0, 0)),
    )
    scores, labels, oboxes, indices = pl.pallas_call(
        _body,
        grid=grid,
        in_specs=in_specs,
        out_specs=out_specs,
        out_shape=out_shapes,
        compiler_params=pltpu.CompilerParams(
            dimension_semantics=("arbitrary",),
        ),
    )(scale, logits, boxes)

    return (
        scores.reshape(B, N),
        labels.reshape(B, N),
        oboxes.reshape(B, N, 4),
        indices.reshape(B, N),
    )


# speedup vs baseline: 1.3668x; 1.3668x over previous
import functools

import jax
import jax.numpy as jnp
from jax.experimental import pallas as pl
from jax.experimental.pallas import tpu as pltpu

B = 8
N = 20000
C = 92
BN = 2000


def _body(scale_ref, logit_ref, box_ref, score_ref, label_ref, obox_ref,
          idx_ref):
    b = pl.program_id(0)
    i = pl.program_id(1)

    x = logit_ref[0]
    m_all = jnp.max(x, axis=-1, keepdims=True)
    cols = jax.lax.broadcasted_iota(jnp.int32, (BN, C), 1)
    valid = cols < (C - 1)
    xm = jnp.where(valid, x, -jnp.inf)
    m91 = jnp.max(xm, axis=-1, keepdims=True)
    lbl = jnp.min(jnp.where(xm == m91, cols, C), axis=-1)
    denom = jnp.sum(jnp.exp(x - m_all), axis=-1, keepdims=True)
    score = jnp.exp(m91 - m_all) / denom

    score_ref[0, 0] = score.reshape(1, BN)
    label_ref[0, 0] = lbl.reshape(1, BN).astype(jnp.int32)

    w_s = scale_ref[b, 0]
    h_s = scale_ref[b, 1]
    wh_cols = jax.lax.broadcasted_iota(jnp.int32, (BN, 2), 1)
    wh = jnp.where(wh_cols == 0, w_s, h_s)
    bx = box_ref[0]
    ctr = bx[:, 0:2]
    half = 0.5 * bx[:, 2:4]
    lo = (ctr - half) * wh
    hi = (ctr + half) * wh
    obox_ref[0, 0] = jnp.concatenate([lo, hi], axis=-1)

    base = i * BN
    idx_ref[0, 0] = (base + jax.lax.broadcasted_iota(
        jnp.int32, (1, BN), 1)).astype(jnp.float32)


def kernel(pred_logits, pred_boxes, target_sizes):
    nb = N // BN
    ts = target_sizes.astype(jnp.float32)
    img_h = ts[:, 0]
    img_w = ts[:, 1]
    scale = jnp.stack([img_w, img_h, img_w, img_h], axis=1)

    grid = (B, nb)
    out_shapes = (
        jax.ShapeDtypeStruct((B, nb, 1, BN), jnp.float32),
        jax.ShapeDtypeStruct((B, nb, 1, BN), jnp.int32),
        jax.ShapeDtypeStruct((B, nb, BN, 4), jnp.float32),
        jax.ShapeDtypeStruct((B, nb, 1, BN), jnp.float32),
    )
    in_specs = [
        pl.BlockSpec(memory_space=pltpu.SMEM),
        pl.BlockSpec((1, BN, C), lambda b, i: (b, i, 0)),
        pl.BlockSpec((1, BN, 4), lambda b, i: (b, i, 0)),
    ]
    out_specs = (
        pl.BlockSpec((1, 1, 1, BN), lambda b, i: (b, i, 0, 0)),
        pl.BlockSpec((1, 1, 1, BN), lambda b, i: (b, i, 0, 0)),
        pl.BlockSpec((1, 1, BN, 4), lambda b, i: (b, i, 0, 0)),
        pl.BlockSpec((1, 1, 1, BN), lambda b, i: (b, i, 0, 0)),
    )
    scores, labels, oboxes, indices = pl.pallas_call(
        _body,
        grid=grid,
        in_specs=in_specs,
        out_specs=out_specs,
        out_shape=out_shapes,
        compiler_params=pltpu.CompilerParams(
            dimension_semantics=("parallel", "arbitrary"),
        ),
    )(scale, pred_logits, pred_boxes)

    return (
        scores.reshape(B, N),
        labels.reshape(B, N),
        oboxes.reshape(B, N, 4),
        indices.reshape(B, N),
    )


# speedup vs baseline: 2.1039x; 1.5393x over previous
import functools

import jax
import jax.numpy as jnp
from jax.experimental import pallas as pl
from jax.experimental.pallas import tpu as pltpu

B = 8
N = 20000
C = 92
BN = 2000


def _body(scale_ref, logit_ref, box_ref, score_ref, label_ref, obox_ref,
          idx_ref):
    b = pl.program_id(0)
    i = pl.program_id(1)

    x = logit_ref[0]
    xt = x.T
    m_all = jnp.max(xt, axis=0, keepdims=True)
    rows = jax.lax.broadcasted_iota(jnp.int32, (C, BN), 0)
    valid = rows < (C - 1)
    xm = jnp.where(valid, xt, -jnp.inf)
    m91 = jnp.max(xm, axis=0, keepdims=True)
    lbl = jnp.min(jnp.where(xm == m91, rows, C), axis=0, keepdims=True)
    denom = jnp.sum(jnp.exp(xt - m_all), axis=0, keepdims=True)
    score = jnp.exp(m91 - m_all) / denom

    score_ref[0, 0] = score
    label_ref[0, 0] = lbl.astype(jnp.int32)

    w_s = scale_ref[b, 0]
    h_s = scale_ref[b, 1]
    wh_cols = jax.lax.broadcasted_iota(jnp.int32, (BN, 2), 1)
    wh = jnp.where(wh_cols == 0, w_s, h_s)
    bx = box_ref[0]
    ctr = bx[:, 0:2]
    half = 0.5 * bx[:, 2:4]
    lo = (ctr - half) * wh
    hi = (ctr + half) * wh
    obox_ref[0, 0] = jnp.concatenate([lo, hi], axis=-1)

    base = i * BN
    idx_ref[0, 0] = (base + jax.lax.broadcasted_iota(
        jnp.int32, (1, BN), 1)).astype(jnp.float32)


def kernel(pred_logits, pred_boxes, target_sizes):
    nb = N // BN
    ts = target_sizes.astype(jnp.float32)
    img_h = ts[:, 0]
    img_w = ts[:, 1]
    scale = jnp.stack([img_w, img_h, img_w, img_h], axis=1)

    grid = (B, nb)
    out_shapes = (
        jax.ShapeDtypeStruct((B, nb, 1, BN), jnp.float32),
        jax.ShapeDtypeStruct((B, nb, 1, BN), jnp.int32),
        jax.ShapeDtypeStruct((B, nb, BN, 4), jnp.float32),
        jax.ShapeDtypeStruct((B, nb, 1, BN), jnp.float32),
    )
    in_specs = [
        pl.BlockSpec(memory_space=pltpu.SMEM),
        pl.BlockSpec((1, BN, C), lambda b, i: (b, i, 0)),
        pl.BlockSpec((1, BN, 4), lambda b, i: (b, i, 0)),
    ]
    out_specs = (
        pl.BlockSpec((1, 1, 1, BN), lambda b, i: (b, i, 0, 0)),
        pl.BlockSpec((1, 1, 1, BN), lambda b, i: (b, i, 0, 0)),
        pl.BlockSpec((1, 1, BN, 4), lambda b, i: (b, i, 0, 0)),
        pl.BlockSpec((1, 1, 1, BN), lambda b, i: (b, i, 0, 0)),
    )
    scores, labels, oboxes, indices = pl.pallas_call(
        _body,
        grid=grid,
        in_specs=in_specs,
        out_specs=out_specs,
        out_shape=out_shapes,
        compiler_params=pltpu.CompilerParams(
            dimension_semantics=("parallel", "arbitrary"),
        ),
    )(scale, pred_logits, pred_boxes)

    return (
        scores.reshape(B, N),
        labels.reshape(B, N),
        oboxes.reshape(B, N, 4),
        indices.reshape(B, N),
    )


# speedup vs baseline: 21.4549x; 10.1978x over previous
import jax
import jax.numpy as jnp
from jax.experimental import pallas as pl
from jax.experimental.pallas import tpu as pltpu

B = 8
N = 20000
C = 92
BN = 2048


def _body(scale_ref, logit_ref, box_ref, score_ref, label_ref, obox_ref,
          idx_ref):
    i = pl.program_id(0)

    m91 = logit_ref[0]
    lbl = jnp.zeros((B, BN), jnp.int32)
    for c in range(1, C - 1):
        xc = logit_ref[c]
        gt = xc > m91
        m91 = jnp.where(gt, xc, m91)
        lbl = jnp.where(gt, c, lbl)
    m_all = jnp.maximum(m91, logit_ref[C - 1])

    denom = jnp.zeros((B, BN), jnp.float32)
    for c in range(C):
        denom = denom + jnp.exp(logit_ref[c] - m_all)

    score_ref[...] = jnp.exp(m91 - m_all) / denom
    label_ref[...] = lbl

    ws = scale_ref[:, 0:1]
    hs = scale_ref[:, 1:2]
    cx = box_ref[:, 0]
    cy = box_ref[:, 1]
    hw = 0.5 * box_ref[:, 2]
    hh = 0.5 * box_ref[:, 3]
    obox_ref[:, 0] = (cx - hw) * ws
    obox_ref[:, 1] = (cy - hh) * hs
    obox_ref[:, 2] = (cx + hw) * ws
    obox_ref[:, 3] = (cy + hh) * hs

    idx_ref[...] = (i * BN + jax.lax.broadcasted_iota(
        jnp.int32, (B, BN), 1)).astype(jnp.float32)


def kernel(pred_logits, pred_boxes, target_sizes):
    logits_t = jnp.transpose(pred_logits, (2, 0, 1))
    boxes_t = jnp.transpose(pred_boxes, (0, 2, 1))
    ts = target_sizes.astype(jnp.float32)
    img_h = ts[:, 0]
    img_w = ts[:, 1]
    scale = jnp.stack([img_w, img_h, img_w, img_h], axis=1)

    grid = (pl.cdiv(N, BN),)
    out_shapes = (
        jax.ShapeDtypeStruct((B, N), jnp.float32),
        jax.ShapeDtypeStruct((B, N), jnp.int32),
        jax.ShapeDtypeStruct((B, 4, N), jnp.float32),
        jax.ShapeDtypeStruct((B, N), jnp.float32),
    )
    in_specs = [
        pl.BlockSpec((B, 4), lambda i: (0, 0)),
        pl.BlockSpec((C, B, BN), lambda i: (0, 0, i)),
        pl.BlockSpec((B, 4, BN), lambda i: (0, 0, i)),
    ]
    out_specs = (
        pl.BlockSpec((B, BN), lambda i: (0, i)),
        pl.BlockSpec((B, BN), lambda i: (0, i)),
        pl.BlockSpec((B, 4, BN), lambda i: (0, 0, i)),
        pl.BlockSpec((B, BN), lambda i: (0, i)),
    )
    scores, labels, oboxes, indices = pl.pallas_call(
        _body,
        grid=grid,
        in_specs=in_specs,
        out_specs=out_specs,
        out_shape=out_shapes,
        compiler_params=pltpu.CompilerParams(
            dimension_semantics=("arbitrary",),
        ),
    )(scale, logits_t, boxes_t)

    return (
        scores,
        labels,
        jnp.transpose(oboxes, (0, 2, 1)),
        indices,
    )
